# baseline (speedup 1.0000x reference)
import functools

import jax
import jax.numpy as jnp
from jax import lax
from jax.experimental import pallas as pl
from jax.experimental.pallas import tpu as pltpu
from jax.experimental.pallas import tpu_sc as plsc

_NC = 2
_NS = 16
_NW = _NC * _NS
_C = 128


def _table_body(charge_ref, temp_ref, emb_ref, w1_ref, w2_ref, wp_ref, out_ref):
    B = charge_ref.shape[0]
    V, D = emb_ref.shape
    charge = charge_ref[...]
    onehot = (charge == lax.broadcasted_iota(jnp.int32, (B, V), 1)).astype(jnp.float32)
    e_charge = jnp.dot(onehot, emb_ref[...], preferred_element_type=jnp.float32)
    t = temp_ref[...]
    h = t * w1_ref[...]
    h = h * jax.nn.sigmoid(h)
    e_temp = jnp.dot(h, w2_ref[...], preferred_element_type=jnp.float32)
    z = jnp.dot(e_charge, wp_ref[:D, :], preferred_element_type=jnp.float32)
    z = z + jnp.dot(e_temp, wp_ref[D:, :], preferred_element_type=jnp.float32)
    out_ref[...] = z * jax.nn.sigmoid(z)


def _make_table(charge2d, temp2d, emb_charge, W1, W2, Wp):
    B = charge2d.shape[0]
    P = Wp.shape[1]
    return pl.pallas_call(
        _table_body,
        out_shape=jax.ShapeDtypeStruct((B, P), jnp.float32),
    )(charge2d, temp2d, emb_charge, W1, W2, Wp)


def _make_gather(N, k_per_w, n_chunks, D, P):
    mesh = plsc.VectorSubcoreMesh(
        core_axis_name="c", subcore_axis_name="s",
        num_cores=_NC, num_subcores=_NS,
    )

    @functools.partial(
        pl.kernel,
        out_type=jax.ShapeDtypeStruct((N, D), jnp.float32),
        mesh=mesh,
        scratch_types=[
            pltpu.VMEM((k_per_w, _C), jnp.int32),
            pltpu.VMEM((_C, P), jnp.float32),
            pltpu.VMEM((_C, P), jnp.float32),
            pltpu.VMEM((_C, D), jnp.float32),
            pltpu.VMEM((_C, D), jnp.float32),
            pltpu.SemaphoreType.DMA,
            pltpu.SemaphoreType.DMA,
            pltpu.SemaphoreType.DMA,
            pltpu.SemaphoreType.DMA,
        ],
        compiler_params=pltpu.CompilerParams(use_tc_tiling_on_sc=True),
    )
    def gather_kernel(table_hbm, idx_hbm, out_hbm, idx_v, big0, big1,
                      nar0, nar1, sem_g0, sem_g1, sem_w0, sem_w1):
        wid = lax.axis_index("s") * _NC + lax.axis_index("c")
        c0 = wid * k_per_w
        nfull = jnp.clip(n_chunks - c0, 0, k_per_w)
        pltpu.sync_copy(idx_hbm.at[wid], idx_v)

        def out_off(j):
            return jnp.minimum((c0 + j) * _C, N - _C)

        def fire_gather(j, big, sem):
            pltpu.async_copy(table_hbm.at[idx_v.at[j]], big, sem)

        def fire_write(j, nar, sem):
            pltpu.async_copy(nar, out_hbm.at[pl.ds(out_off(j), _C)], sem)

        def wait_gather(big, sem):
            pltpu.make_async_copy(table_hbm.at[pl.ds(0, _C)], big, sem).wait()

        def wait_write(nar, sem):
            pltpu.make_async_copy(nar, out_hbm.at[pl.ds(0, _C)], sem).wait()

        def lane_copy(big, nar):
            @pl.loop(0, _C, unroll=8)
            def _row(r):
                for k in range(D // 16):
                    nar[r, pl.ds(16 * k, 16)] = big[r, pl.ds(16 * k, 16)]

        @pl.when(nfull > 0)
        def _():
            fire_gather(0, big0, sem_g0)

        @pl.loop(0, nfull)
        def _chunk(j):
            even = (j % 2) == 0

            @pl.when((j + 1 < nfull) & even)
            def _():
                fire_gather(j + 1, big1, sem_g1)

            @pl.when((j + 1 < nfull) & jnp.logical_not(even))
            def _():
                fire_gather(j + 1, big0, sem_g0)

            @pl.when(even)
            def _():
                wait_gather(big0, sem_g0)

                @pl.when(j >= 2)
                def _():
                    wait_write(nar0, sem_w0)
                lane_copy(big0, nar0)
                fire_write(j, nar0, sem_w0)

            @pl.when(jnp.logical_not(even))
            def _():
                wait_gather(big1, sem_g1)

                @pl.when(j >= 2)
                def _():
                    wait_write(nar1, sem_w1)
                lane_copy(big1, nar1)
                fire_write(j, nar1, sem_w1)

        @pl.when(nfull >= 1)
        def _():
            wait_write(nar0, sem_w0)

        @pl.when(nfull >= 2)
        def _():
            wait_write(nar1, sem_w1)

    return gather_kernel


def kernel(batch, charge, temperature, emb_charge, W1, W2, Wp):
    N = batch.shape[0]
    B = temperature.shape[0]
    D = emb_charge.shape[1]

    P = 128
    Wp_pad = jnp.pad(Wp, ((0, 0), (0, P - D)))
    table = _make_table(
        charge.astype(jnp.int32).reshape(B, 1),
        temperature.reshape(B, 1),
        emb_charge, W1, W2, Wp_pad,
    )

    n_chunks = -(-N // _C)
    k_per_w = -(-n_chunks // _NW)
    batch = batch.astype(jnp.int32)
    idx = jnp.pad(batch, (0, _NW * k_per_w * _C - N)).reshape(_NW, k_per_w, _C)
    last = n_chunks - 1
    idx = idx.at[last // k_per_w, last % k_per_w].set(batch[N - _C:])
    return _make_gather(N, k_per_w, n_chunks, D, P)(table, idx)


# speedup vs baseline: 1.0351x; 1.0351x over previous
import functools

import jax
import jax.numpy as jnp
from jax import lax
from jax.experimental import pallas as pl
from jax.experimental.pallas import tpu as pltpu
from jax.experimental.pallas import tpu_sc as plsc

_NC = 2
_NS = 16
_NW = _NC * _NS
_C = 128


def _table_body(charge_ref, temp_ref, emb_ref, w1_ref, w2_ref, wp_ref, out_ref):
    B = charge_ref.shape[0]
    V, D = emb_ref.shape
    charge = charge_ref[...]
    onehot = (charge == lax.broadcasted_iota(jnp.int32, (B, V), 1)).astype(jnp.float32)
    e_charge = jnp.dot(onehot, emb_ref[...], preferred_element_type=jnp.float32)
    t = temp_ref[...]
    h = t * w1_ref[...]
    h = h * jax.nn.sigmoid(h)
    e_temp = jnp.dot(h, w2_ref[...], preferred_element_type=jnp.float32)
    z = jnp.dot(e_charge, wp_ref[:D, :], preferred_element_type=jnp.float32)
    z = z + jnp.dot(e_temp, wp_ref[D:, :], preferred_element_type=jnp.float32)
    out_ref[...] = z * jax.nn.sigmoid(z)


def _make_table(charge2d, temp2d, emb_charge, W1, W2, Wp):
    B = charge2d.shape[0]
    P = Wp.shape[1]
    return pl.pallas_call(
        _table_body,
        out_shape=jax.ShapeDtypeStruct((B, P), jnp.float32),
    )(charge2d, temp2d, emb_charge, W1, W2, Wp)


def _make_gather(N, k_per_w, n_chunks, D, P):
    mesh = plsc.VectorSubcoreMesh(
        core_axis_name="c", subcore_axis_name="s",
        num_cores=_NC, num_subcores=_NS,
    )

    @functools.partial(
        pl.kernel,
        out_type=jax.ShapeDtypeStruct((N, D), jnp.float32),
        mesh=mesh,
        scratch_types=[
            pltpu.VMEM((k_per_w, _C), jnp.int32),
            pltpu.VMEM((_C, P), jnp.float32),
            pltpu.VMEM((_C, P), jnp.float32),
            pltpu.VMEM((_C, D), jnp.float32),
            pltpu.VMEM((_C, D), jnp.float32),
            pltpu.SemaphoreType.DMA,
            pltpu.SemaphoreType.DMA,
            pltpu.SemaphoreType.DMA,
            pltpu.SemaphoreType.DMA,
        ],
        compiler_params=pltpu.CompilerParams(use_tc_tiling_on_sc=True),
    )
    def gather_kernel(table_hbm, idx_hbm, out_hbm, idx_v, big0, big1,
                      nar0, nar1, sem_g0, sem_g1, sem_w0, sem_w1):
        wid = lax.axis_index("s") * _NC + lax.axis_index("c")
        c0 = wid * k_per_w
        nfull = jnp.clip(n_chunks - c0, 0, k_per_w)
        pltpu.sync_copy(idx_hbm.at[wid], idx_v)

        def out_off(j):
            return jnp.minimum((c0 + j) * _C, N - _C)

        def fire_gather(j, big, sem):
            pltpu.async_copy(table_hbm.at[idx_v.at[j]], big, sem)

        def fire_write(j, nar, sem):
            pltpu.async_copy(nar, out_hbm.at[pl.ds(out_off(j), _C)], sem)

        def wait_gather(big, sem):
            pltpu.make_async_copy(table_hbm.at[pl.ds(0, _C)], big, sem).wait()

        def wait_write(nar, sem):
            pltpu.make_async_copy(nar, out_hbm.at[pl.ds(0, _C)], sem).wait()

        def lane_copy(big, nar):
            nv = D // 16

            @pl.loop(0, _C // 2, unroll=4)
            def _pair(p):
                r = 2 * p
                vals = [big[r + k // nv, pl.ds(16 * (k % nv), 16)]
                        for k in range(2 * nv)]
                for k in range(2 * nv):
                    nar[r + k // nv, pl.ds(16 * (k % nv), 16)] = vals[k]

        @pl.when(nfull > 0)
        def _():
            fire_gather(0, big0, sem_g0)

        @pl.loop(0, nfull)
        def _chunk(j):
            even = (j % 2) == 0

            @pl.when((j + 1 < nfull) & even)
            def _():
                fire_gather(j + 1, big1, sem_g1)

            @pl.when((j + 1 < nfull) & jnp.logical_not(even))
            def _():
                fire_gather(j + 1, big0, sem_g0)

            @pl.when(even)
            def _():
                wait_gather(big0, sem_g0)

                @pl.when(j >= 2)
                def _():
                    wait_write(nar0, sem_w0)
                lane_copy(big0, nar0)
                fire_write(j, nar0, sem_w0)

            @pl.when(jnp.logical_not(even))
            def _():
                wait_gather(big1, sem_g1)

                @pl.when(j >= 2)
                def _():
                    wait_write(nar1, sem_w1)
                lane_copy(big1, nar1)
                fire_write(j, nar1, sem_w1)

        @pl.when(nfull >= 1)
        def _():
            wait_write(nar0, sem_w0)

        @pl.when(nfull >= 2)
        def _():
            wait_write(nar1, sem_w1)

    return gather_kernel


def kernel(batch, charge, temperature, emb_charge, W1, W2, Wp):
    N = batch.shape[0]
    B = temperature.shape[0]
    D = emb_charge.shape[1]

    P = 128
    Wp_pad = jnp.pad(Wp, ((0, 0), (0, P - D)))
    table = _make_table(
        charge.astype(jnp.int32).reshape(B, 1),
        temperature.reshape(B, 1),
        emb_charge, W1, W2, Wp_pad,
    )

    n_chunks = -(-N // _C)
    k_per_w = -(-n_chunks // _NW)
    batch = batch.astype(jnp.int32)
    idx = jnp.pad(batch, (0, _NW * k_per_w * _C - N)).reshape(_NW, k_per_w, _C)
    last = n_chunks - 1
    idx = idx.at[last // k_per_w, last % k_per_w].set(batch[N - _C:])
    return _make_gather(N, k_per_w, n_chunks, D, P)(table, idx)


# speedup vs baseline: 2.3163x; 2.2377x over previous
import functools

import jax
import jax.numpy as jnp
from jax import lax
from jax.experimental import pallas as pl
from jax.experimental.pallas import tpu as pltpu
from jax.experimental.pallas import tpu_sc as plsc

_NC = 2
_NS = 16
_NW = _NC * _NS
_C = 128


def _table_body(charge_ref, temp_ref, emb_ref, w1_ref, w2_ref, wp_ref, out_ref):
    B = charge_ref.shape[0]
    V, D = emb_ref.shape
    charge = charge_ref[...]
    onehot = (charge == lax.broadcasted_iota(jnp.int32, (B, V), 1)).astype(jnp.float32)
    e_charge = jnp.dot(onehot, emb_ref[...], preferred_element_type=jnp.float32)
    t = temp_ref[...]
    h = t * w1_ref[...]
    h = h * jax.nn.sigmoid(h)
    e_temp = jnp.dot(h, w2_ref[...], preferred_element_type=jnp.float32)
    z = jnp.dot(e_charge, wp_ref[:D, :], preferred_element_type=jnp.float32)
    z = z + jnp.dot(e_temp, wp_ref[D:, :], preferred_element_type=jnp.float32)
    out_ref[...] = z * jax.nn.sigmoid(z)


def _make_table(charge2d, temp2d, emb_charge, W1, W2, Wp):
    B = charge2d.shape[0]
    D = emb_charge.shape[1]
    return pl.pallas_call(
        _table_body,
        out_shape=jax.ShapeDtypeStruct((B, D), jnp.float32),
    )(charge2d, temp2d, emb_charge, W1, W2, Wp)


def _make_expand(N, k_per_w, n_chunks, D, B):
    mesh = plsc.VectorSubcoreMesh(
        core_axis_name="c", subcore_axis_name="s",
        num_cores=_NC, num_subcores=_NS,
    )
    rows2 = B // 2
    nv = D // 16

    @functools.partial(
        pl.kernel,
        out_type=jax.ShapeDtypeStruct((N, D), jnp.float32),
        mesh=mesh,
        scratch_types=[
            pltpu.VMEM((rows2, 2 * D), jnp.float32),
            pltpu.VMEM((k_per_w, _C), jnp.int32),
            pltpu.VMEM((_C, D), jnp.float32),
            pltpu.VMEM((_C, D), jnp.float32),
            pltpu.VMEM((_C, D), jnp.float32),
            pltpu.SemaphoreType.DMA,
            pltpu.SemaphoreType.DMA,
            pltpu.SemaphoreType.DMA,
            pltpu.SemaphoreType.DMA,
            pltpu.SemaphoreType.DMA,
        ],
        compiler_params=pltpu.CompilerParams(use_tc_tiling_on_sc=True),
    )
    def expand_kernel(table_hbm, idx_hbm, out_hbm, table_v, idx_v,
                      nar0, nar1, nar2, isem0, isem1, wsem0, wsem1, wsem2):
        nars = (nar0, nar1, nar2)
        wsems = (wsem0, wsem1, wsem2)
        isems = (isem0, isem1)
        wid = lax.axis_index("s") * _NC + lax.axis_index("c")
        c0 = wid * k_per_w
        nfull = jnp.clip(n_chunks - c0, 0, k_per_w)
        pltpu.async_copy(idx_hbm.at[wid], idx_v, isem0)
        pltpu.sync_copy(table_hbm, table_v)
        pltpu.make_async_copy(idx_hbm.at[0], idx_v, isem0).wait()

        def out_off(j):
            return jnp.minimum((c0 + j) * _C, N - _C)


        def fire_write(j, b):
            pltpu.async_copy(nars[b], out_hbm.at[pl.ds(out_off(j), _C)], wsems[b])

        def wait_write(b):
            pltpu.make_async_copy(nars[b], out_hbm.at[pl.ds(0, _C)], wsems[b]).wait()

        def fill(j, b):
            nar = nars[b]

            @pl.loop(0, _C // 16)
            def _group(g):
                idx16 = idx_v[j, pl.ds(16 * g, 16)]
                for h in range(16):
                    bsys = idx16[h]
                    row = lax.shift_right_logical(bsys, 1)
                    colb = lax.mul(lax.bitwise_and(bsys, 1), D)
                    vals = [table_v[row, pl.ds(colb + 16 * k, 16)]
                            for k in range(nv)]
                    for k in range(nv):
                        nar[16 * g + h, pl.ds(16 * k, 16)] = vals[k]

        @pl.loop(0, nfull)
        def _chunk(j):
            b3 = j % 3

            for b in range(3):
                @pl.when(b3 == b)
                def _(b=b):
                    @pl.when(j >= 3)
                    def _():
                        wait_write(b)
                    fill(j, b)
                    fire_write(j, b)

        for b in range(3):
            @pl.when(nfull >= b + 1)
            def _(b=b):
                wait_write(b)

    return expand_kernel


def kernel(batch, charge, temperature, emb_charge, W1, W2, Wp):
    N = batch.shape[0]
    B = temperature.shape[0]
    D = emb_charge.shape[1]

    table = _make_table(
        charge.astype(jnp.int32).reshape(B, 1),
        temperature.reshape(B, 1),
        emb_charge, W1, W2, Wp,
    )
    table2 = table.reshape(B // 2, 2 * D)

    n_chunks = -(-N // _C)
    k_per_w = -(-n_chunks // _NW)
    batch = batch.astype(jnp.int32)
    idx = jnp.pad(batch, (0, _NW * k_per_w * _C - N)).reshape(_NW, k_per_w, _C)
    last = n_chunks - 1
    idx = idx.at[last // k_per_w, last % k_per_w].set(batch[N - _C:])
    return _make_expand(N, k_per_w, n_chunks, D, B)(table2, idx)


# speedup vs baseline: 2.3496x; 1.0144x over previous
import functools

import jax
import jax.numpy as jnp
from jax import lax
from jax.experimental import pallas as pl
from jax.experimental.pallas import tpu as pltpu
from jax.experimental.pallas import tpu_sc as plsc

_NC = 2
_NS = 16
_NW = _NC * _NS
_C = 128


def _table_body(charge_ref, temp_ref, emb_ref, w1_ref, w2_ref, wp_ref, out_ref):
    B = charge_ref.shape[0]
    V, D = emb_ref.shape
    charge = charge_ref[...]
    onehot = (charge == lax.broadcasted_iota(jnp.int32, (B, V), 1)).astype(jnp.float32)
    e_charge = jnp.dot(onehot, emb_ref[...], preferred_element_type=jnp.float32)
    t = temp_ref[...]
    h = t * w1_ref[...]
    h = h * jax.nn.sigmoid(h)
    e_temp = jnp.dot(h, w2_ref[...], preferred_element_type=jnp.float32)
    z = jnp.dot(e_charge, wp_ref[:D, :], preferred_element_type=jnp.float32)
    z = z + jnp.dot(e_temp, wp_ref[D:, :], preferred_element_type=jnp.float32)
    out_ref[...] = z * jax.nn.sigmoid(z)


def _make_table(charge2d, temp2d, emb_charge, W1, W2, Wp):
    B = charge2d.shape[0]
    D = emb_charge.shape[1]
    return pl.pallas_call(
        _table_body,
        out_shape=jax.ShapeDtypeStruct((B, D), jnp.float32),
    )(charge2d, temp2d, emb_charge, W1, W2, Wp)


def _make_expand(N, k_per_w, n_chunks, D, B):
    mesh = plsc.VectorSubcoreMesh(
        core_axis_name="c", subcore_axis_name="s",
        num_cores=_NC, num_subcores=_NS,
    )
    rows2 = B // 2
    nv = D // 16

    @functools.partial(
        pl.kernel,
        out_type=jax.ShapeDtypeStruct((N, D), jnp.float32),
        mesh=mesh,
        scratch_types=[
            pltpu.VMEM((rows2, 2 * D), jnp.float32),
            pltpu.VMEM((k_per_w * _C,), jnp.int32),
            pltpu.VMEM((_C, D), jnp.float32),
            pltpu.VMEM((_C, D), jnp.float32),
            pltpu.VMEM((_C, D), jnp.float32),
            pltpu.SemaphoreType.DMA,
            pltpu.SemaphoreType.DMA,
            pltpu.SemaphoreType.DMA,
            pltpu.SemaphoreType.DMA,
            pltpu.SemaphoreType.DMA,
        ],
        compiler_params=pltpu.CompilerParams(use_tc_tiling_on_sc=True),
    )
    def expand_kernel(table_hbm, idx_hbm, out_hbm, table_v, idx_v,
                      nar0, nar1, nar2, isem0, isem1, wsem0, wsem1, wsem2):
        nars = (nar0, nar1, nar2)
        wsems = (wsem0, wsem1, wsem2)
        isems = (isem0, isem1)
        wid = lax.axis_index("s") * _NC + lax.axis_index("c")
        c0 = wid * k_per_w
        nfull = jnp.clip(n_chunks - c0, 0, k_per_w)
        base = jnp.minimum(c0 * _C, N - k_per_w * _C)
        pltpu.async_copy(idx_hbm.at[pl.ds(base, k_per_w * _C)], idx_v, isem0)
        pltpu.sync_copy(table_hbm, table_v)
        pltpu.make_async_copy(idx_hbm.at[pl.ds(0, k_per_w * _C)], idx_v, isem0).wait()

        def out_off(j):
            return jnp.minimum((c0 + j) * _C, N - _C)


        def fire_write(j, b):
            pltpu.async_copy(nars[b], out_hbm.at[pl.ds(out_off(j), _C)], wsems[b])

        def wait_write(b):
            pltpu.make_async_copy(nars[b], out_hbm.at[pl.ds(0, _C)], wsems[b]).wait()

        def fill(j, b):
            nar = nars[b]
            ibase = out_off(j) - base

            @pl.loop(0, _C // 16)
            def _group(g):
                idx16 = idx_v[pl.ds(ibase + 16 * g, 16)]
                for h in range(16):
                    bsys = idx16[h]
                    row = lax.shift_right_logical(bsys, 1)
                    colb = lax.mul(lax.bitwise_and(bsys, 1), D)
                    vals = [table_v[row, pl.ds(colb + 16 * k, 16)]
                            for k in range(nv)]
                    for k in range(nv):
                        nar[16 * g + h, pl.ds(16 * k, 16)] = vals[k]

        @pl.loop(0, nfull)
        def _chunk(j):
            b3 = j % 3

            for b in range(3):
                @pl.when(b3 == b)
                def _(b=b):
                    @pl.when(j >= 3)
                    def _():
                        wait_write(b)
                    fill(j, b)
                    fire_write(j, b)

        for b in range(3):
            @pl.when(nfull >= b + 1)
            def _(b=b):
                wait_write(b)

    return expand_kernel


def kernel(batch, charge, temperature, emb_charge, W1, W2, Wp):
    N = batch.shape[0]
    B = temperature.shape[0]
    D = emb_charge.shape[1]

    table = _make_table(
        charge.astype(jnp.int32).reshape(B, 1),
        temperature.reshape(B, 1),
        emb_charge, W1, W2, Wp,
    )
    table2 = table.reshape(B // 2, 2 * D)

    n_chunks = -(-N // _C)
    k_per_w = -(-n_chunks // _NW)
    return _make_expand(N, k_per_w, n_chunks, D, B)(table2, batch.astype(jnp.int32))
